# baseline (speedup 1.0000x reference)
import jax
import jax.numpy as jnp
from jax.experimental import pallas as pl
from jax.experimental.pallas import tpu as pltpu
from jax.experimental.pallas import tpu_sc as plsc

_GROUPS = 8
_POOL = 256
_TOPK = 8
_L = 12
_R = 4
_H = 768
_GH = 96
_BATCH = 1024

_HIGH = jax.lax.Precision.HIGHEST


def _sim_topk_body(x_ref, k_ref, dw_ref, idx_ref):
    x = x_ref[...]
    ns = jnp.sum(x * x, axis=2, keepdims=True)
    inv = jax.lax.rsqrt(jnp.maximum(ns, 1e-16))
    qsum = jnp.sum(x * inv, axis=0)

    k3 = k_ref[0]
    kns = jnp.sum(k3 * k3, axis=2, keepdims=True)
    kn = k3 * jax.lax.rsqrt(jnp.maximum(kns, 1e-16))

    sim = jnp.zeros((1, _POOL), jnp.float32)
    for g in range(_GROUPS):
        qg = qsum[g:g + 1, :]
        sim = sim + jax.lax.dot_general(
            qg, kn[g], (((1,), (1,)), ((), ())),
            precision=_HIGH, preferred_element_type=jnp.float32)

    v = sim / float(_BATCH * _GROUPS)
    lane = jax.lax.broadcasted_iota(jnp.int32, (1, _POOL), 1)
    vals, idxs = [], []
    for _ in range(_TOPK):
        m = jnp.max(v)
        ii = jnp.min(jnp.where(v == m, lane, _POOL))
        vals.append(m)
        idxs.append(ii)
        v = jnp.where(lane == ii, -jnp.inf, v)
    total = vals[0]
    for k in range(1, _TOPK):
        total = total + vals[k]
    denom = total + 1e-9
    for k in range(_TOPK):
        dw_ref[k] = vals[k] / denom
        for j in range(8):
            idx_ref[k * 8 + j] = idxs[k]


def _sim_topk(x3, k3):
    return pl.pallas_call(
        _sim_topk_body,
        in_specs=[pl.BlockSpec(memory_space=pltpu.VMEM),
                  pl.BlockSpec(memory_space=pltpu.VMEM)],
        out_specs=(pl.BlockSpec(memory_space=pltpu.SMEM),
                   pl.BlockSpec(memory_space=pltpu.SMEM)),
        out_shape=(jax.ShapeDtypeStruct((_TOPK,), jnp.float32),
                   jax.ShapeDtypeStruct((_TOPK * 8,), jnp.int32)),
    )(x3, k3)


def _sc_gather(wo5, idx):
    mesh = plsc.VectorSubcoreMesh(core_axis_name="c", subcore_axis_name="s")
    row = _R * _H * _L
    units = 2 * 16
    slices_per_row = units // (_TOPK * 2)
    chunk = row // slices_per_row

    @pl.kernel(
        out_type=jax.ShapeDtypeStruct((2 * _TOPK, row), jnp.float32),
        mesh=mesh,
        scratch_types=[pltpu.VMEM((_TOPK * 8,), jnp.int32),
                       pltpu.VMEM((1, 2, row), jnp.float32),
                       pltpu.SemaphoreType.DMA,
                       pltpu.SemaphoreType.DMA],
    )
    def gather_kernel(wo_ref, idx_ref, g_ref, idx_vmem, row_vmem, sem_i,
                      sem_d):
        pltpu.async_copy(idx_ref, idx_vmem, sem_i).wait()
        c = jax.lax.axis_index("c")
        s = jax.lax.axis_index("s")
        u = c * 16 + s

        @pl.when(u < _TOPK)
        def _():
            iv = idx_vmem.at[pl.ds(u * 8, 1)]
            pltpu.async_copy(wo_ref.at[iv], row_vmem, sem_d).wait()
            ca = pltpu.async_copy(
                row_vmem.at[pl.ds(0, 1), 0], g_ref.at[pl.ds(u, 1)], sem_d)
            cb = pltpu.async_copy(
                row_vmem.at[pl.ds(0, 1), 1],
                g_ref.at[pl.ds(_TOPK + u, 1)], sem_d)
            ca.wait()
            cb.wait()

    return gather_kernel(wo5, idx)


def _combine_body(dw_ref, a_ref, b_ref, o_ref):
    ab = a_ref[...]
    bb = b_ref[...]
    a = jnp.concatenate(
        [ab[:, r * _H:(r + 1) * _H] for r in range(_R)], axis=0)
    b = jnp.concatenate(
        [bb[:, r * _H:(r + 1) * _H] for r in range(_R)], axis=0)
    rows = jax.lax.broadcasted_iota(jnp.int32, (_TOPK * _R, _H), 0) % _TOPK
    w = jnp.zeros((_TOPK * _R, _H), jnp.float32)
    for n in range(_TOPK):
        w = jnp.where(rows == n, dw_ref[n], w)
    j = pl.program_id(1)
    bh = jnp.where(j == 0, b[:, :_H // 2], b[:, _H // 2:])
    o_ref[0] = jax.lax.dot_general(
        a * w, bh, (((0,), (0,)), ((), ())),
        preferred_element_type=jnp.float32)


def _combine(dw, g):
    return pl.pallas_call(
        _combine_body,
        grid=(_L, 2),
        in_specs=[
            pl.BlockSpec(memory_space=pltpu.SMEM),
            pl.BlockSpec((_TOPK, _R * _H), lambda l, j: (0, l)),
            pl.BlockSpec((_TOPK, _R * _H), lambda l, j: (1, l)),
        ],
        out_specs=pl.BlockSpec((1, _H, _H // 2), lambda l, j: (l, 0, j)),
        out_shape=jax.ShapeDtypeStruct((_L, _H, _H), jnp.float32),
        compiler_params=pltpu.CompilerParams(
            dimension_semantics=("parallel", "parallel")),
    )(dw, g, g)


def kernel(inputs, keys, weight_offset):
    x3 = inputs.reshape(_BATCH, _GROUPS, _GH)
    dw, idx = _sim_topk(x3, keys)
    g = _sc_gather(weight_offset, idx)
    return _combine(dw, g)


# speedup vs baseline: 1.2023x; 1.2023x over previous
import jax
import jax.numpy as jnp
from jax.experimental import pallas as pl
from jax.experimental.pallas import tpu as pltpu
from jax.experimental.pallas import tpu_sc as plsc

_GROUPS = 8
_POOL = 256
_TOPK = 8
_L = 12
_R = 4
_H = 768
_GH = 96
_BATCH = 1024

_HIGH = jax.lax.Precision.HIGHEST


def _sim_topk_body(x_ref, k_ref, dw_ref, idx_ref):
    x = x_ref[...]
    seg = jax.lax.broadcasted_iota(jnp.int32, (_GROUPS * _GH, _GROUPS), 0)
    gid = jax.lax.broadcasted_iota(jnp.int32, (_GROUPS * _GH, _GROUPS), 1)
    s_mat = (seg // _GH == gid).astype(jnp.float32)
    ns = jax.lax.dot_general(
        x * x, s_mat, (((1,), (0,)), ((), ())),
        preferred_element_type=jnp.float32)
    inv = jax.lax.rsqrt(jnp.maximum(ns, 1e-16))
    scale = jax.lax.dot_general(
        inv, s_mat, (((1,), (1,)), ((), ())),
        preferred_element_type=jnp.float32)
    qsum = jnp.sum(x * scale, axis=0, keepdims=True)

    k3 = k_ref[0]
    kns = jnp.sum(k3 * k3, axis=2, keepdims=True)
    kn = k3 * jax.lax.rsqrt(jnp.maximum(kns, 1e-16))

    sim = jnp.zeros((1, _POOL), jnp.float32)
    for g in range(_GROUPS):
        qg = qsum[:, g * _GH:(g + 1) * _GH]
        sim = sim + jax.lax.dot_general(
            qg, kn[g], (((1,), (1,)), ((), ())),
            precision=_HIGH, preferred_element_type=jnp.float32)

    v = sim / float(_BATCH * _GROUPS)
    lane = jax.lax.broadcasted_iota(jnp.int32, (1, _POOL), 1)
    vals, idxs = [], []
    for _ in range(_TOPK):
        m = jnp.max(v)
        ii = jnp.min(jnp.where(v == m, lane, _POOL))
        vals.append(m)
        idxs.append(ii)
        v = jnp.where(lane == ii, -jnp.inf, v)
    total = vals[0]
    for k in range(1, _TOPK):
        total = total + vals[k]
    denom = total + 1e-9
    for k in range(_TOPK):
        dw_ref[k] = vals[k] / denom
        for j in range(8):
            idx_ref[k * 8 + j] = idxs[k]


def _sim_topk(x3, k3):
    return pl.pallas_call(
        _sim_topk_body,
        in_specs=[pl.BlockSpec(memory_space=pltpu.VMEM),
                  pl.BlockSpec(memory_space=pltpu.VMEM)],
        out_specs=(pl.BlockSpec(memory_space=pltpu.SMEM),
                   pl.BlockSpec(memory_space=pltpu.SMEM)),
        out_shape=(jax.ShapeDtypeStruct((_TOPK,), jnp.float32),
                   jax.ShapeDtypeStruct((_TOPK * 8,), jnp.int32)),
    )(x3, k3)


def _sc_gather(wo5, idx):
    mesh = plsc.VectorSubcoreMesh(core_axis_name="c", subcore_axis_name="s")
    row = _R * _H * _L
    units = 2 * 16
    slices_per_row = units // (_TOPK * 2)
    chunk = row // slices_per_row

    @pl.kernel(
        out_type=jax.ShapeDtypeStruct((2 * _TOPK, row), jnp.float32),
        mesh=mesh,
        scratch_types=[pltpu.VMEM((_TOPK * 8,), jnp.int32),
                       pltpu.VMEM((1, 2, row), jnp.float32),
                       pltpu.SemaphoreType.DMA,
                       pltpu.SemaphoreType.DMA],
    )
    def gather_kernel(wo_ref, idx_ref, g_ref, idx_vmem, row_vmem, sem_i,
                      sem_d):
        pltpu.async_copy(idx_ref, idx_vmem, sem_i).wait()
        c = jax.lax.axis_index("c")
        s = jax.lax.axis_index("s")
        u = c * 16 + s

        @pl.when(u < _TOPK)
        def _():
            iv = idx_vmem.at[pl.ds(u * 8, 1)]
            pltpu.async_copy(wo_ref.at[iv], row_vmem, sem_d).wait()
            ca = pltpu.async_copy(
                row_vmem.at[pl.ds(0, 1), 0], g_ref.at[pl.ds(u, 1)], sem_d)
            cb = pltpu.async_copy(
                row_vmem.at[pl.ds(0, 1), 1],
                g_ref.at[pl.ds(_TOPK + u, 1)], sem_d)
            ca.wait()
            cb.wait()

    return gather_kernel(wo5, idx)


def _combine_body(dw_ref, a_ref, b_ref, o_ref):
    ab = a_ref[...]
    bb = b_ref[...]
    a = jnp.concatenate(
        [ab[:, r * _H:(r + 1) * _H] for r in range(_R)], axis=0)
    b = jnp.concatenate(
        [bb[:, r * _H:(r + 1) * _H] for r in range(_R)], axis=0)
    rows = jax.lax.broadcasted_iota(jnp.int32, (_TOPK * _R, _H), 0) % _TOPK
    w = jnp.zeros((_TOPK * _R, _H), jnp.float32)
    for n in range(_TOPK):
        w = jnp.where(rows == n, dw_ref[n], w)
    o_ref[0] = jax.lax.dot_general(
        a * w, b, (((0,), (0,)), ((), ())),
        preferred_element_type=jnp.float32)


def _combine(dw, g):
    return pl.pallas_call(
        _combine_body,
        grid=(_L,),
        in_specs=[
            pl.BlockSpec(memory_space=pltpu.SMEM),
            pl.BlockSpec((_TOPK, _R * _H), lambda l: (0, l)),
            pl.BlockSpec((_TOPK, _R * _H), lambda l: (1, l)),
        ],
        out_specs=pl.BlockSpec((1, _H, _H), lambda l: (l, 0, 0)),
        out_shape=jax.ShapeDtypeStruct((_L, _H, _H), jnp.float32),
        compiler_params=pltpu.CompilerParams(
            dimension_semantics=("parallel",)),
    )(dw, g, g)


def kernel(inputs, keys, weight_offset):
    dw, idx = _sim_topk(inputs, keys)
    g = _sc_gather(weight_offset, idx)
    return _combine(dw, g)


# speedup vs baseline: 1.2206x; 1.0152x over previous
import jax
import jax.numpy as jnp
from jax.experimental import pallas as pl
from jax.experimental.pallas import tpu as pltpu
from jax.experimental.pallas import tpu_sc as plsc

_GROUPS = 8
_POOL = 256
_TOPK = 8
_L = 12
_R = 4
_H = 768
_GH = 96
_BATCH = 1024

_HIGH = jax.lax.Precision.HIGHEST


def _sim_topk_body(x_ref, k_ref, dw_ref, idx_ref):
    x = x_ref[...]
    seg = jax.lax.broadcasted_iota(jnp.int32, (_GROUPS * _GH, _GROUPS), 0)
    gid = jax.lax.broadcasted_iota(jnp.int32, (_GROUPS * _GH, _GROUPS), 1)
    s_mat = (seg // _GH == gid).astype(jnp.float32)
    ns = jax.lax.dot_general(
        x * x, s_mat, (((1,), (0,)), ((), ())),
        preferred_element_type=jnp.float32)
    inv = jax.lax.rsqrt(jnp.maximum(ns, 1e-16))
    scale = jax.lax.dot_general(
        inv, s_mat, (((1,), (1,)), ((), ())),
        preferred_element_type=jnp.float32)
    qsum = jnp.sum(x * scale, axis=0, keepdims=True)

    k3 = k_ref[0]
    kns = jnp.sum(k3 * k3, axis=2, keepdims=True)
    kn = k3 * jax.lax.rsqrt(jnp.maximum(kns, 1e-16))

    sim = jnp.zeros((1, _POOL), jnp.float32)
    for g in range(_GROUPS):
        qg = qsum[:, g * _GH:(g + 1) * _GH]
        sim = sim + jax.lax.dot_general(
            qg, kn[g], (((1,), (1,)), ((), ())),
            precision=_HIGH, preferred_element_type=jnp.float32)

    v = sim / float(_BATCH * _GROUPS)
    lane = jax.lax.broadcasted_iota(jnp.int32, (1, _POOL), 1)
    vals, idxs = [], []
    for _ in range(_TOPK):
        m = jnp.max(v)
        ii = jnp.min(jnp.where(v == m, lane, _POOL))
        vals.append(m)
        idxs.append(ii)
        v = jnp.where(lane == ii, -jnp.inf, v)
    total = vals[0]
    for k in range(1, _TOPK):
        total = total + vals[k]
    denom = total + 1e-9
    for k in range(_TOPK):
        dw_ref[k] = vals[k] / denom
        for j in range(8):
            idx_ref[k * 8 + j] = idxs[k]


def _sim_topk(x3, k3):
    return pl.pallas_call(
        _sim_topk_body,
        in_specs=[pl.BlockSpec(memory_space=pltpu.VMEM),
                  pl.BlockSpec(memory_space=pltpu.VMEM)],
        out_specs=(pl.BlockSpec(memory_space=pltpu.SMEM),
                   pl.BlockSpec(memory_space=pltpu.SMEM)),
        out_shape=(jax.ShapeDtypeStruct((_TOPK,), jnp.float32),
                   jax.ShapeDtypeStruct((_TOPK * 8,), jnp.int32)),
    )(x3, k3)


def _sc_gather(wo5, idx):
    mesh = plsc.VectorSubcoreMesh(core_axis_name="c", subcore_axis_name="s")
    row = _R * _H * _L
    units = 2 * 16
    slices_per_row = units // (_TOPK * 2)
    chunk = row // slices_per_row

    @pl.kernel(
        out_type=jax.ShapeDtypeStruct((2 * _TOPK, row), jnp.float32),
        mesh=mesh,
        scratch_types=[pltpu.VMEM((_TOPK * 8,), jnp.int32),
                       pltpu.VMEM((1, 2, row), jnp.float32),
                       pltpu.SemaphoreType.DMA,
                       pltpu.SemaphoreType.DMA],
    )
    def gather_kernel(wo_ref, idx_ref, g_ref, idx_vmem, row_vmem, sem_i,
                      sem_d):
        c = jax.lax.axis_index("c")
        s = jax.lax.axis_index("s")
        u = c * 16 + s

        @pl.when(u < _TOPK)
        def _():
            pltpu.async_copy(idx_ref, idx_vmem, sem_i).wait()
            iv = idx_vmem.at[pl.ds(u * 8, 1)]
            pltpu.async_copy(wo_ref.at[iv], row_vmem, sem_d).wait()
            ca = pltpu.async_copy(
                row_vmem.at[pl.ds(0, 1), 0], g_ref.at[pl.ds(u, 1)], sem_d)
            cb = pltpu.async_copy(
                row_vmem.at[pl.ds(0, 1), 1],
                g_ref.at[pl.ds(_TOPK + u, 1)], sem_d)
            ca.wait()
            cb.wait()

    return gather_kernel(wo5, idx)


def _combine_body(dw_ref, a_ref, b_ref, o_ref):
    ab = a_ref[...]
    bb = b_ref[...]
    a = jnp.concatenate(
        [ab[:, r * _H:(r + 1) * _H] for r in range(_R)], axis=0)
    b = jnp.concatenate(
        [bb[:, r * _H:(r + 1) * _H] for r in range(_R)], axis=0)
    rows = jax.lax.broadcasted_iota(jnp.int32, (_TOPK * _R, _H), 0) % _TOPK
    w = jnp.zeros((_TOPK * _R, _H), jnp.float32)
    for n in range(_TOPK):
        w = jnp.where(rows == n, dw_ref[n], w)
    o_ref[0] = jax.lax.dot_general(
        a * w, b, (((0,), (0,)), ((), ())),
        preferred_element_type=jnp.float32)


def _combine(dw, g):
    return pl.pallas_call(
        _combine_body,
        grid=(_L,),
        in_specs=[
            pl.BlockSpec(memory_space=pltpu.SMEM),
            pl.BlockSpec((_TOPK, _R * _H), lambda l: (0, l)),
            pl.BlockSpec((_TOPK, _R * _H), lambda l: (1, l)),
        ],
        out_specs=pl.BlockSpec((1, _H, _H), lambda l: (l, 0, 0)),
        out_shape=jax.ShapeDtypeStruct((_L, _H, _H), jnp.float32),
        compiler_params=pltpu.CompilerParams(
            dimension_semantics=("parallel",)),
    )(dw, g, g)


def kernel(inputs, keys, weight_offset):
    dw, idx = _sim_topk(inputs, keys)
    g = _sc_gather(weight_offset, idx)
    return _combine(dw, g)
